# Initial kernel scaffold; baseline (speedup 1.0000x reference)
import jax
import jax.numpy as jnp
from jax.experimental import pallas as pl


def kernel(x, q, Wk, Wv, Wfc):
    raise NotImplementedError("write your pallas kernel here")


# speedup vs baseline: 1.3366x; 1.3366x over previous
import functools

import jax
import jax.numpy as jnp
import numpy as np
from jax.experimental import pallas as pl
from jax.experimental.pallas import tpu as pltpu

D_MODEL, D_K, D_V, N_Q = 1024, 128, 128, 64
B, T = 4, 2048


def _fused_body(x_ref, q_ref, wk_ref, wv_ref, wfc_ref, out_ref, hard_ref):
    x = x_ref[0]
    k = jax.lax.dot_general(
        x, wk_ref[...], (((1,), (1,)), ((), ())),
        preferred_element_type=jnp.float32)
    attn = jax.lax.dot_general(
        k, q_ref[...], (((1,), (1,)), ((), ())),
        preferred_element_type=jnp.float32) * (1.0 / np.sqrt(N_Q))
    m = jnp.max(attn, axis=-1, keepdims=True)
    iota = jax.lax.broadcasted_iota(jnp.int32, attn.shape, 1)
    idx = jnp.min(jnp.where(attn == m, iota, N_Q), axis=-1, keepdims=True)
    onehot = (iota == idx).astype(jnp.float32)
    hard_ref[0] = onehot
    xsum = jax.lax.dot_general(
        onehot, x, (((0,), (0,)), ((), ())),
        preferred_element_type=jnp.float32)
    vslot = jax.lax.dot_general(
        xsum, wv_ref[...], (((1,), (1,)), ((), ())),
        preferred_element_type=jnp.float32)
    out_ref[0] = jax.lax.dot_general(
        vslot, wfc_ref[...], (((1,), (1,)), ((), ())),
        preferred_element_type=jnp.float32)


@functools.partial(jax.jit, static_argnames=("interpret",))
def kernel(x, q, Wk, Wv, Wfc, interpret=False):
    out, hard = pl.pallas_call(
        _fused_body,
        grid=(B,),
        in_specs=[
            pl.BlockSpec((1, T, D_MODEL), lambda b: (b, 0, 0)),
            pl.BlockSpec((N_Q, D_K), lambda b: (0, 0)),
            pl.BlockSpec((D_K, D_MODEL), lambda b: (0, 0)),
            pl.BlockSpec((D_V, D_MODEL), lambda b: (0, 0)),
            pl.BlockSpec((D_MODEL, D_V), lambda b: (0, 0)),
        ],
        out_specs=[
            pl.BlockSpec((1, N_Q, D_MODEL), lambda b: (b, 0, 0)),
            pl.BlockSpec((1, T, N_Q), lambda b: (b, 0, 0)),
        ],
        out_shape=[
            jax.ShapeDtypeStruct((B, N_Q, D_MODEL), jnp.float32),
            jax.ShapeDtypeStruct((B, T, N_Q), jnp.float32),
        ],
        compiler_params=pltpu.CompilerParams(
            dimension_semantics=("arbitrary",),
        ),
        interpret=interpret,
    )(x, q, Wk, Wv, Wfc)
    return out, hard
